# Initial kernel scaffold; baseline (speedup 1.0000x reference)
import jax
import jax.numpy as jnp
from jax.experimental import pallas as pl


def kernel(Laplacian, X, W1, W2, W3, W4):
    raise NotImplementedError("write your pallas kernel here")


# speedup vs baseline: 1.4670x; 1.4670x over previous
import jax
import jax.numpy as jnp
from jax.experimental import pallas as pl
from jax.experimental.pallas import tpu as pltpu

N = 4096
D_IN, D_MID, D_EMB = 1024, 256, 64
BM = 256
CK = 512
F32 = jnp.float32


def _mm_kernel(x_ref, w_ref, o_ref):
    o_ref[...] = jnp.dot(x_ref[...], w_ref[...], preferred_element_type=F32)


def _enc_kernel(l_ref, a_ref, w2_ref, o_ref):
    h = jnp.maximum(jnp.dot(l_ref[...], a_ref[...], preferred_element_type=F32), 0.0)
    o_ref[...] = jnp.dot(h, w2_ref[...], preferred_element_type=F32)


def _emb_kernel(l_ref, b_ref, e_ref, et_ref):
    e = jnp.dot(l_ref[...], b_ref[...], preferred_element_type=F32)
    e_ref[...] = e
    et_ref[...] = e.T


def _head_kernel(e_ref, et_ref, w3_ref, w4_ref, w_out_ref, x_out_ref):
    e = e_ref[...]
    sq_blk = jnp.sum(e * e, axis=1, keepdims=True)

    def pass1(j, acc):
        etc = et_ref[:, pl.ds(j * CK, CK)]
        g = jnp.dot(e, etc, preferred_element_type=F32)
        sqc = jnp.sum(etc * etc, axis=0, keepdims=True)
        d = jnp.maximum(sq_blk + sqc - 2.0 * g, 0.0)
        p = jnp.exp(-d)
        w_out_ref[:, pl.ds(j * CK, CK)] = p
        return acc + jnp.sum(p, axis=1, keepdims=True)

    s = jax.lax.fori_loop(0, N // CK, pass1, jnp.zeros((BM, 1), F32))
    inv = 1.0 / s

    def pass2(j, _):
        w_out_ref[:, pl.ds(j * CK, CK)] = (
            w_out_ref[:, pl.ds(j * CK, CK)] * inv + 1e-10
        )
        return 0

    jax.lax.fori_loop(0, N // CK, pass2, 0)

    h = jnp.maximum(jnp.dot(e, w3_ref[...], preferred_element_type=F32), 0.0)
    x_out_ref[...] = jnp.dot(h, w4_ref[...], preferred_element_type=F32) + 1e-10


def kernel(Laplacian, X, W1, W2, W3, W4):
    nb = N // BM

    A = pl.pallas_call(
        _mm_kernel,
        grid=(nb,),
        in_specs=[
            pl.BlockSpec((BM, D_IN), lambda i: (i, 0)),
            pl.BlockSpec((D_IN, D_MID), lambda i: (0, 0)),
        ],
        out_specs=pl.BlockSpec((BM, D_MID), lambda i: (i, 0)),
        out_shape=jax.ShapeDtypeStruct((N, D_MID), F32),
    )(X, W1)

    B = pl.pallas_call(
        _enc_kernel,
        grid=(nb,),
        in_specs=[
            pl.BlockSpec((BM, N), lambda i: (i, 0)),
            pl.BlockSpec((N, D_MID), lambda i: (0, 0)),
            pl.BlockSpec((D_MID, D_EMB), lambda i: (0, 0)),
        ],
        out_specs=pl.BlockSpec((BM, D_EMB), lambda i: (i, 0)),
        out_shape=jax.ShapeDtypeStruct((N, D_EMB), F32),
    )(Laplacian, A, W2)

    E, ET = pl.pallas_call(
        _emb_kernel,
        grid=(nb,),
        in_specs=[
            pl.BlockSpec((BM, N), lambda i: (i, 0)),
            pl.BlockSpec((N, D_EMB), lambda i: (0, 0)),
        ],
        out_specs=[
            pl.BlockSpec((BM, D_EMB), lambda i: (i, 0)),
            pl.BlockSpec((D_EMB, BM), lambda i: (0, i)),
        ],
        out_shape=[
            jax.ShapeDtypeStruct((N, D_EMB), F32),
            jax.ShapeDtypeStruct((D_EMB, N), F32),
        ],
    )(Laplacian, B)

    recons_w, recons_x = pl.pallas_call(
        _head_kernel,
        grid=(nb,),
        in_specs=[
            pl.BlockSpec((BM, D_EMB), lambda i: (i, 0)),
            pl.BlockSpec((D_EMB, N), lambda i: (0, 0)),
            pl.BlockSpec((D_EMB, D_MID), lambda i: (0, 0)),
            pl.BlockSpec((D_MID, D_IN), lambda i: (0, 0)),
        ],
        out_specs=[
            pl.BlockSpec((BM, N), lambda i: (i, 0)),
            pl.BlockSpec((BM, D_IN), lambda i: (i, 0)),
        ],
        out_shape=[
            jax.ShapeDtypeStruct((N, N), F32),
            jax.ShapeDtypeStruct((N, D_IN), F32),
        ],
    )(E, ET, W3, W4)

    return (recons_w, recons_x)


# speedup vs baseline: 1.4878x; 1.0142x over previous
import jax
import jax.numpy as jnp
from jax.experimental import pallas as pl
from jax.experimental.pallas import tpu as pltpu

N = 4096
D_IN, D_MID, D_EMB = 1024, 256, 64
BM = 256
CK = 512
F32 = jnp.float32


BF16 = jnp.bfloat16


def _mm_kernel(x_ref, w_ref, o_ref):
    o_ref[...] = jnp.dot(
        x_ref[...].astype(BF16), w_ref[...].astype(BF16),
        preferred_element_type=F32,
    ).astype(BF16)


def _enc_kernel(l_ref, a_ref, w2_ref, o_ref):
    h = jnp.maximum(
        jnp.dot(l_ref[...].astype(BF16), a_ref[...], preferred_element_type=F32),
        0.0,
    )
    o_ref[...] = jnp.dot(
        h.astype(BF16), w2_ref[...].astype(BF16), preferred_element_type=F32
    ).astype(BF16)


def _emb_kernel(l_ref, b_ref, e_ref, et_ref):
    e = jnp.dot(l_ref[...].astype(BF16), b_ref[...], preferred_element_type=F32)
    e_ref[...] = e
    et_ref[...] = e.T


def _head_kernel(e_ref, et_ref, w3_ref, w4_ref, w_out_ref, x_out_ref):
    e = e_ref[...]
    sq_blk = jnp.sum(e * e, axis=1, keepdims=True)

    def pass1(j, acc):
        etc = et_ref[:, pl.ds(j * CK, CK)]
        g = jnp.dot(e, etc, preferred_element_type=F32)
        sqc = jnp.sum(etc * etc, axis=0, keepdims=True)
        d = jnp.maximum(sq_blk + sqc - 2.0 * g, 0.0)
        p = jnp.exp(-d)
        w_out_ref[:, pl.ds(j * CK, CK)] = p
        return acc + jnp.sum(p, axis=1, keepdims=True)

    s = jax.lax.fori_loop(0, N // CK, pass1, jnp.zeros((BM, 1), F32))
    inv = 1.0 / s

    def pass2(j, _):
        w_out_ref[:, pl.ds(j * CK, CK)] = (
            w_out_ref[:, pl.ds(j * CK, CK)] * inv + 1e-10
        )
        return 0

    jax.lax.fori_loop(0, N // CK, pass2, 0)

    h = jnp.maximum(jnp.dot(e, w3_ref[...], preferred_element_type=F32), 0.0)
    x_out_ref[...] = jnp.dot(h, w4_ref[...], preferred_element_type=F32) + 1e-10


def kernel(Laplacian, X, W1, W2, W3, W4):
    nb = N // BM

    A = pl.pallas_call(
        _mm_kernel,
        grid=(nb,),
        in_specs=[
            pl.BlockSpec((BM, D_IN), lambda i: (i, 0)),
            pl.BlockSpec((D_IN, D_MID), lambda i: (0, 0)),
        ],
        out_specs=pl.BlockSpec((BM, D_MID), lambda i: (i, 0)),
        out_shape=jax.ShapeDtypeStruct((N, D_MID), BF16),
    )(X, W1)

    B = pl.pallas_call(
        _enc_kernel,
        grid=(nb,),
        in_specs=[
            pl.BlockSpec((BM, N), lambda i: (i, 0)),
            pl.BlockSpec((N, D_MID), lambda i: (0, 0)),
            pl.BlockSpec((D_MID, D_EMB), lambda i: (0, 0)),
        ],
        out_specs=pl.BlockSpec((BM, D_EMB), lambda i: (i, 0)),
        out_shape=jax.ShapeDtypeStruct((N, D_EMB), BF16),
    )(Laplacian, A, W2)

    E, ET = pl.pallas_call(
        _emb_kernel,
        grid=(nb,),
        in_specs=[
            pl.BlockSpec((BM, N), lambda i: (i, 0)),
            pl.BlockSpec((N, D_EMB), lambda i: (0, 0)),
        ],
        out_specs=[
            pl.BlockSpec((BM, D_EMB), lambda i: (i, 0)),
            pl.BlockSpec((D_EMB, BM), lambda i: (0, i)),
        ],
        out_shape=[
            jax.ShapeDtypeStruct((N, D_EMB), F32),
            jax.ShapeDtypeStruct((D_EMB, N), F32),
        ],
    )(Laplacian, B)

    recons_w, recons_x = pl.pallas_call(
        _head_kernel,
        grid=(nb,),
        in_specs=[
            pl.BlockSpec((BM, D_EMB), lambda i: (i, 0)),
            pl.BlockSpec((D_EMB, N), lambda i: (0, 0)),
            pl.BlockSpec((D_EMB, D_MID), lambda i: (0, 0)),
            pl.BlockSpec((D_MID, D_IN), lambda i: (0, 0)),
        ],
        out_specs=[
            pl.BlockSpec((BM, N), lambda i: (i, 0)),
            pl.BlockSpec((BM, D_IN), lambda i: (i, 0)),
        ],
        out_shape=[
            jax.ShapeDtypeStruct((N, N), F32),
            jax.ShapeDtypeStruct((N, D_IN), F32),
        ],
    )(E, ET, W3, W4)

    return (recons_w, recons_x)
